# baseline (speedup 1.0000x reference)
import functools

import jax
import jax.numpy as jnp
from jax.experimental import pallas as pl

_K_BLK = 12544


def _scan_kernel(x_ref, c_ref, vals_ref, idxs_ref, *, k_total, k_blk, nb):
    j = pl.program_id(0)
    nc = k_blk // 128
    n = x_ref.shape[0]

    x = x_ref[...]
    x2 = x * 2.0
    xsq = jnp.sum(x * x, axis=1, keepdims=True)

    c = c_ref[...]
    cn = jnp.sum(c * c, axis=0, keepdims=True)
    col = jax.lax.broadcasted_iota(jnp.int32, (1, k_blk), 1) + j * k_blk
    cn = jnp.where(col < k_total, cn, jnp.inf)

    m2 = jax.lax.dot_general(
        x2, c, (((1,), (0,)), ((), ())), preferred_element_type=jnp.float32
    )

    bval = (xsq - m2[:, 0:128]) + cn[:, 0:128]
    bidx = jnp.zeros((n, 128), jnp.int32)
    for i in range(1, nc):
        sl = slice(i * 128, (i + 1) * 128)
        di = (xsq - m2[:, sl]) + cn[:, sl]
        lt = di < bval
        bval = jnp.minimum(di, bval)
        bidx = jnp.where(lt, i, bidx)
    lane = jax.lax.broadcasted_iota(jnp.int32, (n, 128), 1)
    gidx = bidx * 128 + lane + j * k_blk

    @pl.when(j == 0)
    def _():
        vals_ref[...] = bval
        idxs_ref[...] = gidx

    @pl.when(j > 0)
    def _():
        rv = vals_ref[...]
        lt = bval < rv
        vals_ref[...] = jnp.where(lt, bval, rv)
        idxs_ref[...] = jnp.where(lt, gidx, idxs_ref[...])


def _merge_kernel(vals_ref, idxs_ref, out_ref):
    fv = vals_ref[...]
    fi = idxs_ref[...]
    gmin = jnp.min(fv, axis=1, keepdims=True)
    cand = jnp.where(fv == gmin, fi, jnp.int32(2**31 - 1))
    res = jnp.min(cand, axis=1)
    out_ref[...] = res.reshape(out_ref.shape)


@jax.jit
def kernel(x, centroids):
    n, d = x.shape
    k_total = centroids.shape[1]
    nb = -(-k_total // _K_BLK)
    kp = nb * _K_BLK
    cpad = jnp.pad(centroids, ((0, 0), (0, kp - k_total)))

    vals, idxs = pl.pallas_call(
        functools.partial(_scan_kernel, k_total=k_total, k_blk=_K_BLK, nb=nb),
        grid=(nb,),
        in_specs=[
            pl.BlockSpec((n, d), lambda j: (0, 0)),
            pl.BlockSpec((d, _K_BLK), lambda j: (0, j)),
        ],
        out_specs=[
            pl.BlockSpec((n, 128), lambda j: (0, 0)),
            pl.BlockSpec((n, 128), lambda j: (0, 0)),
        ],
        out_shape=[
            jax.ShapeDtypeStruct((n, 128), jnp.float32),
            jax.ShapeDtypeStruct((n, 128), jnp.int32),
        ],
    )(x, cpad)

    out = pl.pallas_call(
        _merge_kernel,
        out_shape=jax.ShapeDtypeStruct((n // 128, 128), jnp.int32),
    )(vals, idxs)
    return out.reshape(-1)


# speedup vs baseline: 1.0095x; 1.0095x over previous
import functools

import jax
import jax.numpy as jnp
from jax.experimental import pallas as pl
from jax.experimental.pallas import tpu as pltpu

_K_BLK = 12544


def _argmin_kernel(x_ref, c_ref, out_ref, vals_ref, idxs_ref, *, k_total, k_blk, nb):
    j = pl.program_id(0)
    nc = k_blk // 128
    n = x_ref.shape[0]

    x = x_ref[...]
    x2 = x * 2.0
    xsq = jnp.sum(x * x, axis=1, keepdims=True)

    c = c_ref[:, pl.ds(j * k_blk, k_blk)]
    cn = jnp.sum(c * c, axis=0, keepdims=True)
    col = jax.lax.broadcasted_iota(jnp.int32, (1, k_blk), 1) + j * k_blk
    cn = jnp.where(col < k_total, cn, jnp.inf)

    m2 = jax.lax.dot_general(
        x2, c, (((1,), (0,)), ((), ())), preferred_element_type=jnp.float32
    )

    bval = (xsq - m2[:, 0:128]) + cn[:, 0:128]
    bidx = jnp.zeros((n, 128), jnp.int32)
    for i in range(1, nc):
        sl = slice(i * 128, (i + 1) * 128)
        di = (xsq - m2[:, sl]) + cn[:, sl]
        lt = di < bval
        bval = jnp.minimum(di, bval)
        bidx = jnp.where(lt, i, bidx)
    lane = jax.lax.broadcasted_iota(jnp.int32, (n, 128), 1)
    gidx = bidx * 128 + lane + j * k_blk

    @pl.when(j == 0)
    def _():
        vals_ref[...] = bval
        idxs_ref[...] = gidx

    @pl.when(j > 0)
    def _():
        rv = vals_ref[...]
        lt = bval < rv
        vals_ref[...] = jnp.where(lt, bval, rv)
        idxs_ref[...] = jnp.where(lt, gidx, idxs_ref[...])

    @pl.when(j == nb - 1)
    def _():
        fv = vals_ref[...]
        fi = idxs_ref[...]
        gmin = jnp.min(fv, axis=1, keepdims=True)
        cand = jnp.where(fv == gmin, fi, jnp.int32(2**31 - 1))
        res = jnp.min(cand, axis=1)
        out_ref[...] = res.reshape(out_ref.shape)


@jax.jit
def kernel(x, centroids):
    n, d = x.shape
    k_total = centroids.shape[1]
    nb = -(-k_total // _K_BLK)
    kp = nb * _K_BLK
    cpad = jnp.pad(centroids, ((0, 0), (0, kp - k_total)))

    out = pl.pallas_call(
        functools.partial(_argmin_kernel, k_total=k_total, k_blk=_K_BLK, nb=nb),
        grid=(nb,),
        in_specs=[
            pl.BlockSpec((n, d), lambda j: (0, 0)),
            pl.BlockSpec((d, kp), lambda j: (0, 0)),
        ],
        out_specs=pl.BlockSpec((n // 128, 128), lambda j: (0, 0)),
        out_shape=jax.ShapeDtypeStruct((n // 128, 128), jnp.int32),
        scratch_shapes=[
            pltpu.VMEM((n, 128), jnp.float32),
            pltpu.VMEM((n, 128), jnp.int32),
        ],
    )(x, cpad)
    return out.reshape(-1)


# speedup vs baseline: 1.0275x; 1.0179x over previous
import functools

import jax
import jax.numpy as jnp
from jax.experimental import pallas as pl
from jax.experimental.pallas import tpu as pltpu

_K_BLK = 12544


def _argmin_kernel(x_ref, c_ref, out_ref, vals_ref, idxs_ref, *, k_total, k_blk, nb):
    j = pl.program_id(0)
    nc = k_blk // 128
    n = x_ref.shape[0]

    x = x_ref[...]
    x2 = x * 2.0
    xsq = jnp.sum(x * x, axis=1, keepdims=True)

    c = c_ref[...]
    cn = jnp.sum(c * c, axis=0, keepdims=True)
    col = jax.lax.broadcasted_iota(jnp.int32, (1, k_blk), 1) + j * k_blk
    cn = jnp.where(col < k_total, cn, jnp.inf)

    m2 = jax.lax.dot_general(
        x2, c, (((1,), (0,)), ((), ())), preferred_element_type=jnp.float32
    )

    bval = (xsq - m2[:, 0:128]) + cn[:, 0:128]
    bidx = jnp.zeros((n, 128), jnp.int32)
    for i in range(1, nc):
        sl = slice(i * 128, (i + 1) * 128)
        di = (xsq - m2[:, sl]) + cn[:, sl]
        lt = di < bval
        bval = jnp.minimum(di, bval)
        bidx = jnp.where(lt, i, bidx)
    lane = jax.lax.broadcasted_iota(jnp.int32, (n, 128), 1)
    gidx = bidx * 128 + lane + j * k_blk

    @pl.when(j == 0)
    def _():
        vals_ref[...] = bval
        idxs_ref[...] = gidx

    @pl.when(j > 0)
    def _():
        rv = vals_ref[...]
        lt = bval < rv
        vals_ref[...] = jnp.where(lt, bval, rv)
        idxs_ref[...] = jnp.where(lt, gidx, idxs_ref[...])

    @pl.when(j == nb - 1)
    def _():
        fv = vals_ref[...]
        fi = idxs_ref[...]
        gmin = jnp.min(fv, axis=1, keepdims=True)
        cand = jnp.where(fv == gmin, fi, jnp.int32(2**31 - 1))
        res = jnp.min(cand, axis=1)
        out_ref[...] = res.reshape(out_ref.shape)


@jax.jit
def kernel(x, centroids):
    n, d = x.shape
    k_total = centroids.shape[1]
    nb = -(-k_total // _K_BLK)
    kp = nb * _K_BLK
    cpad = jnp.pad(centroids, ((0, 0), (0, kp - k_total)))

    out = pl.pallas_call(
        functools.partial(_argmin_kernel, k_total=k_total, k_blk=_K_BLK, nb=nb),
        grid=(nb,),
        in_specs=[
            pl.BlockSpec((n, d), lambda j: (0, 0)),
            pl.BlockSpec((d, _K_BLK), lambda j: (0, j)),
        ],
        out_specs=pl.BlockSpec((n // 128, 128), lambda j: (0, 0)),
        out_shape=jax.ShapeDtypeStruct((n // 128, 128), jnp.int32),
        scratch_shapes=[
            pltpu.VMEM((n, 128), jnp.float32),
            pltpu.VMEM((n, 128), jnp.int32),
        ],
    )(x, cpad)
    return out.reshape(-1)
